# baseline (speedup 1.0000x reference)
import functools

import jax
import jax.numpy as jnp
from jax import lax
from jax.experimental import pallas as pl
from jax.experimental.pallas import tpu as pltpu
from jax.experimental.pallas import tpu_sc as plsc

_EMB_DIM = 64
_SEQ_LEN = 200
_NC = 2
_NS = 16
_NW = _NC * _NS
_CHUNK = 128
_LANES = 16


def _pos_embedding(emb_dim, seq_len):
    pos = jnp.arange(seq_len, dtype=jnp.float32)[:, None]
    i = jnp.arange(emb_dim, dtype=jnp.float32)[None, :]
    angle_rates = 1.0 / jnp.power(10000.0, (2.0 * jnp.floor(i / 2.0)) / float(emb_dim))
    angles = pos * angle_rates
    even = (jnp.arange(emb_dim)[None, :] % 2) == 0
    return jnp.where(even, jnp.sin(angles), jnp.cos(angles)).astype(jnp.float32)


def kernel(inputs, token_embeddings):
    batch, seq_len = inputs.shape
    _, emb_dim = token_embeddings.shape
    rows = batch * seq_len
    rpw = rows // _NW
    nch = rpw // _CHUNK
    nvec = emb_dim // _LANES

    pe = _pos_embedding(emb_dim, seq_len)
    pe2 = jnp.concatenate([pe, pe], axis=0)
    idx = inputs.reshape(_NW, nch, _CHUNK)

    mesh = plsc.VectorSubcoreMesh(core_axis_name="c", subcore_axis_name="s")

    @functools.partial(
        pl.kernel,
        out_type=jax.ShapeDtypeStruct((rows, emb_dim), jnp.float32),
        mesh=mesh,
        compiler_params=pltpu.CompilerParams(use_tc_tiling_on_sc=False),
        scratch_types=[
            pltpu.VMEM((nch, _CHUNK), jnp.int32),
            pltpu.VMEM((2 * seq_len, emb_dim), jnp.float32),
            pltpu.VMEM((_CHUNK, emb_dim), jnp.float32),
            pltpu.VMEM((_CHUNK, emb_dim), jnp.float32),
            pltpu.SemaphoreType.DMA,
            pltpu.SemaphoreType.DMA,
        ],
    )
    def emb(idx_hbm, table_hbm, pe_hbm, out_hbm,
            idx_v, pe_v, rows0, rows1, sem0, sem1):
        wid = lax.axis_index("s") * _NC + lax.axis_index("c")
        base = wid * rpw
        pltpu.sync_copy(idx_hbm.at[wid], idx_v)
        pltpu.sync_copy(pe_hbm, pe_v)

        def issue_gather(j, buf, sem):
            pltpu.async_copy(table_hbm.at[idx_v.at[j]], buf, sem)

        def wait_gather(j, buf, sem):
            pltpu.make_async_copy(table_hbm.at[idx_v.at[j]], buf, sem).wait()

        def add_store(j, buf):
            p0 = lax.rem(j * _CHUNK, seq_len)

            def row_body(r, carry):
                for c in range(nvec):
                    sl = pl.ds(c * _LANES, _LANES)
                    plsc.addupdate(buf.at[r, sl], pe_v[p0 + r, sl])
                return carry

            lax.fori_loop(0, _CHUNK, row_body, 0, unroll=2)
            pltpu.sync_copy(buf, out_hbm.at[pl.ds(base + j * _CHUNK, _CHUNK)])

        issue_gather(0, rows0, sem0)
        issue_gather(1, rows1, sem1)

        def body(jj, carry):
            j0 = 2 * jj
            j1 = j0 + 1
            wait_gather(j0, rows0, sem0)
            add_store(j0, rows0)

            @pl.when(j0 + 2 < nch)
            def _():
                issue_gather(j0 + 2, rows0, sem0)

            wait_gather(j1, rows1, sem1)
            add_store(j1, rows1)

            @pl.when(j1 + 2 < nch)
            def _():
                issue_gather(j1 + 2, rows1, sem1)

            return carry

        lax.fori_loop(0, nch // 2, body, 0)

    out = emb(idx, token_embeddings, pe2)
    return out.reshape(batch, seq_len, emb_dim)


# speedup vs baseline: 1.0605x; 1.0605x over previous
import functools

import jax
import jax.numpy as jnp
from jax import lax
from jax.experimental import pallas as pl
from jax.experimental.pallas import tpu as pltpu
from jax.experimental.pallas import tpu_sc as plsc

_EMB_DIM = 64
_SEQ_LEN = 200
_NC = 2
_NS = 16
_NW = _NC * _NS
_CHUNK = 128
_LANES = 16
_NB = 4
_RUNROLL = 16


def _pos_embedding(emb_dim, seq_len):
    pos = jnp.arange(seq_len, dtype=jnp.float32)[:, None]
    i = jnp.arange(emb_dim, dtype=jnp.float32)[None, :]
    angle_rates = 1.0 / jnp.power(10000.0, (2.0 * jnp.floor(i / 2.0)) / float(emb_dim))
    angles = pos * angle_rates
    even = (jnp.arange(emb_dim)[None, :] % 2) == 0
    return jnp.where(even, jnp.sin(angles), jnp.cos(angles)).astype(jnp.float32)


def kernel(inputs, token_embeddings):
    batch, seq_len = inputs.shape
    _, emb_dim = token_embeddings.shape
    rows = batch * seq_len
    rpw = rows // _NW
    nch = rpw // _CHUNK
    nvec = emb_dim // _LANES

    pe = _pos_embedding(emb_dim, seq_len)
    pe2 = jnp.concatenate([pe, pe], axis=0)
    idx = inputs.reshape(_NW, nch, _CHUNK)

    mesh = plsc.VectorSubcoreMesh(core_axis_name="c", subcore_axis_name="s")

    @functools.partial(
        pl.kernel,
        out_type=jax.ShapeDtypeStruct((rows, emb_dim), jnp.float32),
        mesh=mesh,
        compiler_params=pltpu.CompilerParams(use_tc_tiling_on_sc=False),
        scratch_types=[
            pltpu.VMEM((nch, _CHUNK), jnp.int32),
            pltpu.VMEM((2 * seq_len, emb_dim), jnp.float32),
            [pltpu.VMEM((_CHUNK, emb_dim), jnp.float32) for _ in range(_NB)],
            [pltpu.SemaphoreType.DMA for _ in range(_NB)],
            [pltpu.SemaphoreType.DMA for _ in range(_NB)],
        ],
    )
    def emb(idx_hbm, table_hbm, pe_hbm, out_hbm,
            idx_v, pe_v, bufs, gsems, ssems):
        wid = lax.axis_index("s") * _NC + lax.axis_index("c")
        base = wid * rpw
        pltpu.sync_copy(idx_hbm.at[wid], idx_v)
        pltpu.sync_copy(pe_hbm, pe_v)

        def issue_gather(j, b):
            pltpu.async_copy(table_hbm.at[idx_v.at[j]], bufs[b], gsems[b])

        def wait_gather(j, b):
            pltpu.make_async_copy(
                table_hbm.at[idx_v.at[j]], bufs[b], gsems[b]).wait()

        def out_slice(j):
            return out_hbm.at[pl.ds(base + j * _CHUNK, _CHUNK)]

        def issue_store(j, b):
            pltpu.async_copy(bufs[b], out_slice(j), ssems[b])

        def wait_store(j, b):
            pltpu.make_async_copy(bufs[b], out_slice(j), ssems[b]).wait()

        def add_pe(j, b):
            p0 = lax.rem(j * _CHUNK, seq_len)
            buf = bufs[b]

            def row_body(r0, carry):
                rbase = r0 * _RUNROLL
                for u in range(_RUNROLL):
                    for c in range(nvec):
                        sl = pl.ds(c * _LANES, _LANES)
                        plsc.addupdate(buf.at[rbase + u, sl],
                                       pe_v[p0 + rbase + u, sl])
                return carry

            lax.fori_loop(0, _CHUNK // _RUNROLL, row_body, 0)

        for b in range(_NB - 2):
            issue_gather(b, b)

        def body(jj, carry):
            for k in range(_NB):
                j = _NB * jj + k
                bq = (k - 2) % _NB

                @pl.when(j >= 2)
                def _():
                    wait_store(j - 2, bq)

                @pl.when(j + _NB - 2 < nch)
                def _():
                    issue_gather(j + _NB - 2, bq)

                wait_gather(j, k)
                add_pe(j, k)
                issue_store(j, k)
            return carry

        lax.fori_loop(0, nch // _NB, body, 0)

        for b in (_NB - 2, _NB - 1):
            wait_store(nch - _NB + b, b)

    out = emb(idx, token_embeddings, pe2)
    return out.reshape(batch, seq_len, emb_dim)
